# baseline (speedup 1.0000x reference)
import functools

import jax
import jax.numpy as jnp
from jax import lax
from jax.experimental import pallas as pl
from jax.experimental.pallas import tpu as pltpu
from jax.experimental.pallas import tpu_sc as plsc

_GAMMA = 12.0
_GAMMA_TYPE = 6.0
_HID = 64
_TDIM = 32
_C = 128
_NIDX = 3


@functools.cache
def _build(B):
  info = plsc.get_sparse_core_info()
  NC, NS, L = info.num_cores, info.num_subcores, info.num_lanes
  NW = NC * NS
  assert B % (NW * _C) == 0
  per_w = B // NW
  n_chunks = per_w // _C
  assert n_chunks % 2 == 0
  groups = _C // L
  f32 = jnp.float32
  i32 = jnp.int32
  mesh = plsc.VectorSubcoreMesh(core_axis_name="c", subcore_axis_name="s")

  def _set():
    return [
        pltpu.VMEM((_NIDX * _C,), i32),
        pltpu.VMEM((2 * _C, _HID), f32),
        pltpu.VMEM((2 * _C, _TDIM), f32),
        pltpu.VMEM((_C, _HID), f32),
        pltpu.VMEM((_C, _HID), f32),
        pltpu.VMEM((_C, _TDIM), f32),
        pltpu.VMEM((_C, _TDIM), f32),
        pltpu.SemaphoreType.DMA,
    ]

  @functools.partial(
      pl.kernel,
      mesh=mesh,
      compiler_params=pltpu.CompilerParams(
          needs_layout_passes=False,
          use_tc_tiling_on_sc=False,
          disable_bounds_checks=True,
      ),
      out_type=[jax.ShapeDtypeStruct((B,), f32),
                jax.ShapeDtypeStruct((B,), f32)],
      scratch_types=(
          _set() + _set()
          + [
              pltpu.VMEM((per_w,), f32),
              pltpu.VMEM((per_w,), f32),
          ]),
  )
  def kge(pack_hbm, ent_hbm, rel_hbm, typ_hbm, rtyp_hbm, nv_hbm, nvt_hbm,
          score_hbm, scoret_hbm, *scratch):
    set0 = scratch[0:8]
    set1 = scratch[8:16]
    sc_v, sct_v = scratch[16:18]
    wid = lax.axis_index("s") * NC + lax.axis_index("c")
    base = wid * per_w

    def copies(bufs):
      idx_v, ent_v, typ_v, rel_v, nv_v, rtyp_v, nvt_v, sem = bufs
      ht = idx_v.at[pl.ds(0, 2 * _C)]
      rr = idx_v.at[pl.ds(2 * _C, _C)]
      return [
          pltpu.make_async_copy(ent_hbm.at[ht], ent_v, sem),
          pltpu.make_async_copy(typ_hbm.at[ht], typ_v, sem),
          pltpu.make_async_copy(rel_hbm.at[rr], rel_v, sem),
          pltpu.make_async_copy(nv_hbm.at[rr], nv_v, sem),
          pltpu.make_async_copy(rtyp_hbm.at[rr], rtyp_v, sem),
          pltpu.make_async_copy(nvt_hbm.at[rr], nvt_v, sem),
      ]

    def start_chunk(bufs, ci):
      idx_v = bufs[0]
      gchunk = wid * n_chunks + ci
      pltpu.sync_copy(pack_hbm.at[pl.ds(gchunk * (_NIDX * _C), _NIDX * _C)],
                      idx_v)
      for cp in copies(bufs):
        cp.start()

    def wait_chunk(bufs):
      for cp in copies(bufs):
        cp.wait()

    def compute_chunk(bufs, ci):
      idx_v, ent_v, typ_v, rel_v, nv_v, rtyp_v, nvt_v, _ = bufs
      lane = lax.iota(i32, L)

      def rsum(x):
        return jnp.broadcast_to(jnp.sum(x), (L,))

      def group_body(g, carry):
        score_acc = jnp.zeros((L,), f32)
        scoret_acc = jnp.zeros((L,), f32)
        for k in range(L):
          i = g * L + k

          hs = [ent_v[i, pl.ds(16 * q, 16)] for q in range(4)]
          ts = [ent_v[_C + i, pl.ds(16 * q, 16)] for q in range(4)]
          rs = [rel_v[i, pl.ds(16 * q, 16)] for q in range(4)]
          vs = [nv_v[i, pl.ds(16 * q, 16)] for q in range(4)]
          hv = rsum((hs[0] * vs[0] + hs[1] * vs[1])
                    + (hs[2] * vs[2] + hs[3] * vs[3]))
          tv = rsum((ts[0] * vs[0] + ts[1] * vs[1])
                    + (ts[2] * vs[2] + ts[3] * vs[3]))
          vv = rsum((vs[0] * vs[0] + vs[1] * vs[1])
                    + (vs[2] * vs[2] + vs[3] * vs[3]))
          c = (tv - hv) / jnp.maximum(vv, 1e-24)
          s4 = [jnp.abs(hs[q] + rs[q] - ts[q] + c * vs[q]) for q in range(4)]
          score = _GAMMA - rsum((s4[0] + s4[1]) + (s4[2] + s4[3]))

          h2s = [typ_v[i, pl.ds(16 * q, 16)] for q in range(2)]
          t2s = [typ_v[_C + i, pl.ds(16 * q, 16)] for q in range(2)]
          r2s = [rtyp_v[i, pl.ds(16 * q, 16)] for q in range(2)]
          v2s = [nvt_v[i, pl.ds(16 * q, 16)] for q in range(2)]
          hv2 = rsum(h2s[0] * v2s[0] + h2s[1] * v2s[1])
          tv2 = rsum(t2s[0] * v2s[0] + t2s[1] * v2s[1])
          vv2 = rsum(v2s[0] * v2s[0] + v2s[1] * v2s[1])
          c2 = (tv2 - hv2) / jnp.maximum(vv2, 1e-24)
          s2 = [jnp.abs(h2s[q] + r2s[q] - t2s[q] + c2 * v2s[q])
                for q in range(2)]
          score_t = _GAMMA_TYPE - rsum(s2[0] + s2[1])

          score_acc = jnp.where(lane == k, score, score_acc)
          scoret_acc = jnp.where(lane == k, score_t, scoret_acc)

        out_off = ci * _C + g * L
        sc_v[pl.ds(out_off, L)] = score_acc
        sct_v[pl.ds(out_off, L)] = scoret_acc
        return carry

      lax.fori_loop(0, groups, group_body, 0)

    start_chunk(set0, 0)

    def chunk_pair(ci2, carry):
      ci = ci2 * 2
      wait_chunk(set0)
      start_chunk(set1, ci + 1)
      compute_chunk(set0, ci)
      wait_chunk(set1)

      @pl.when(ci + 2 < n_chunks)
      def _():
        start_chunk(set0, ci + 2)

      compute_chunk(set1, ci + 1)
      return carry

    lax.fori_loop(0, n_chunks // 2, chunk_pair, 0)
    pltpu.sync_copy(sc_v, score_hbm.at[pl.ds(base, per_w)])
    pltpu.sync_copy(sct_v, scoret_hbm.at[pl.ds(base, per_w)])

  return kge


def kernel(sample, entity_embedding, relation_embedding, type_embedding,
           reltype_embedding, norm_vector_embedding, norm_vectortype_embedding):
  B = sample.shape[0]
  fn = _build(B)
  h = sample[:, 0]
  r = sample[:, 1]
  t = sample[:, 2]
  idx3 = jnp.stack([h, t, r])
  pack = idx3.reshape(_NIDX, B // _C, _C).transpose(1, 0, 2).reshape(-1)
  one = (sample[0, 0] * 0 + 1).astype(jnp.float32)
  score, score_type = fn(
      pack, entity_embedding * one, relation_embedding * one,
      type_embedding * one, reltype_embedding * one,
      norm_vector_embedding * one, norm_vectortype_embedding * one)
  return score.reshape(B, 1), score_type.reshape(B, 1)
